# baseline (speedup 1.0000x reference)
import jax
import jax.numpy as jnp
from jax.experimental import pallas as pl

_N0, _N1, _N2 = 10000, 2500, 625


def _feast(x, edge_index, p, n_nodes):
    src, dst = edge_index[0], edge_index[1]
    x_j = x[src]
    x_i = x[dst]
    q = jax.nn.softmax((x_j - x_i) @ p['u'] + p['c'], axis=-1)
    out = jnp.zeros((n_nodes, p['W'].shape[2]), dtype=x.dtype)
    for h in range(p['W'].shape[0]):
        agg = jax.ops.segment_sum(q[:, h:h + 1] * x_j, dst, num_segments=n_nodes)
        out = out + agg @ p['W'][h]
    deg = jax.ops.segment_sum(jnp.ones((src.shape[0],), x.dtype), dst, num_segments=n_nodes)
    out = out / jnp.clip(deg, 1.0, None)[:, None]
    return out + p['b']


def _res(x, edge_index, p, n_nodes, batch_norm=True, relu=True):
    h = _feast(x, edge_index, p, n_nodes)
    if batch_norm:
        mu = jnp.mean(h, axis=0, keepdims=True)
        var = jnp.var(h, axis=0, keepdims=True)
        h = (h - mu) / jnp.sqrt(var + 1e-5) * p['gamma'] + p['beta']
    if relu:
        h = jax.nn.relu(h)
    return h + x @ p['P']


def _pool(x, cluster, n_clusters):
    cnt = jax.ops.segment_sum(jnp.ones((x.shape[0],), x.dtype), cluster, num_segments=n_clusters)
    s = jax.ops.segment_sum(x, cluster, num_segments=n_clusters)
    return s / jnp.clip(cnt, 1.0, None)[:, None]


def _identity_kernel(x_ref, o_ref):
    o_ref[...] = x_ref[...]


def kernel(norm, geo, edge_index0, cluster1, edge_index1, cluster2, edge_index2, params):
    x = jnp.hstack((norm, geo[:, None]))
    x = _res(x, edge_index0, params['conv01'], _N0)
    x = _res(x, edge_index0, params['conv02'], _N0)
    copy0 = x
    x1 = _pool(x, cluster1, _N1)
    x1 = _res(x1, edge_index1, params['conv11'], _N1)
    x1 = _res(x1, edge_index1, params['conv12'], _N1)
    copy1 = x1
    x2 = _pool(x1, cluster2, _N2)
    x2 = _res(x2, edge_index2, params['conv21'], _N2)
    x2 = _res(x2, edge_index2, params['conv22'], _N2)
    x1 = x2[cluster2]
    x1 = jnp.concatenate((x1, copy1), axis=1)
    x1 = _res(x1, edge_index1, params['conv13'], _N1)
    x1 = _res(x1, edge_index1, params['conv14'], _N1)
    x1 = _res(x1, edge_index1, params['conv15'], _N1)
    x1 = _res(x1, edge_index1, params['conv16'], _N1)
    x = x1[cluster1]
    x = jnp.concatenate((x, copy0), axis=1)
    x = _res(x, edge_index0, params['conv03'], _N0)
    x = _res(x, edge_index0, params['conv04'], _N0)
    x = _res(x, edge_index0, params['conv05'], _N0)
    x = _res(x, edge_index0, params['conv06'], _N0, batch_norm=False, relu=False)
    out = pl.pallas_call(
        _identity_kernel,
        out_shape=jax.ShapeDtypeStruct(x.shape, x.dtype),
    )(x)
    return jnp.squeeze(out)


# speedup vs baseline: 1.7932x; 1.7932x over previous
import functools

import jax
import jax.numpy as jnp
from jax import lax
from jax.experimental import pallas as pl
from jax.experimental.pallas import tpu as pltpu
from jax.experimental.pallas import tpu_sc as plsc

_N0, _E0 = 10000, 320000
_N1, _E1 = 2500, 80000
_N2, _E2 = 2500 // 4, 20000
_C, _H = 94, 4

_NC, _NS = 2, 16
_NW = _NC * _NS
_CHUNK = 128
_EALIGN = _NW * _CHUNK
_DX = 128
_DY = 16


def _ceil_to(x, m):
    return (x + m - 1) // m * m


_N0A = _ceil_to(_N0 + 1, 128)
_N1A = _ceil_to(_N1 + 1, 128)
_N2A = _ceil_to(_N2 + 1, 128)


def _sc_gather_two(xtab, srcp, dstp):
    epad = srcp.shape[0]
    width = xtab.shape[1]
    rows_per_tile = epad // _NW
    nchunks = rows_per_tile // _CHUNK
    mesh = plsc.VectorSubcoreMesh(core_axis_name="c", subcore_axis_name="s")

    @functools.partial(
        pl.kernel,
        out_type=(jax.ShapeDtypeStruct((epad, width), jnp.float32),
                  jax.ShapeDtypeStruct((epad, width), jnp.float32)),
        mesh=mesh,
        scratch_types=[pltpu.VMEM((_CHUNK,), jnp.int32),
                       pltpu.VMEM((_CHUNK,), jnp.int32),
                       pltpu.VMEM((_CHUNK, width), jnp.float32),
                       pltpu.VMEM((_CHUNK, width), jnp.float32),
                       pltpu.SemaphoreType.DMA,
                       pltpu.SemaphoreType.DMA],
    )
    def k(x_hbm, src_hbm, dst_hbm, xj_hbm, xi_hbm, si, di, jb, ib, s1, s2):
        wid = lax.axis_index("s") * _NC + lax.axis_index("c")
        base0 = wid * rows_per_tile

        @pl.loop(0, nchunks)
        def _(i):
            base = base0 + i * _CHUNK
            pltpu.sync_copy(src_hbm.at[pl.ds(base, _CHUNK)], si)
            pltpu.sync_copy(dst_hbm.at[pl.ds(base, _CHUNK)], di)
            c1 = pltpu.async_copy(x_hbm.at[si], jb, s1)
            c2 = pltpu.async_copy(x_hbm.at[di], ib, s2)
            c1.wait()
            c2.wait()
            pltpu.sync_copy(jb, xj_hbm.at[pl.ds(base, _CHUNK)])
            pltpu.sync_copy(ib, xi_hbm.at[pl.ds(base, _CHUNK)])

    return k(xtab, srcp, dstp)


def _sc_gather_one(table, idxp):
    epad = idxp.shape[0]
    width = table.shape[1]
    rows_per_tile = epad // _NW
    nchunks = rows_per_tile // _CHUNK
    mesh = plsc.VectorSubcoreMesh(core_axis_name="c", subcore_axis_name="s")

    @functools.partial(
        pl.kernel,
        out_type=jax.ShapeDtypeStruct((epad, width), jnp.float32),
        mesh=mesh,
        scratch_types=[pltpu.VMEM((_CHUNK,), jnp.int32),
                       pltpu.VMEM((_CHUNK, width), jnp.float32),
                       pltpu.SemaphoreType.DMA],
    )
    def k(t_hbm, idx_hbm, o_hbm, ib, rb, sem):
        wid = lax.axis_index("s") * _NC + lax.axis_index("c")
        base0 = wid * rows_per_tile

        @pl.loop(0, nchunks)
        def _(i):
            base = base0 + i * _CHUNK
            pltpu.sync_copy(idx_hbm.at[pl.ds(base, _CHUNK)], ib)
            pltpu.async_copy(t_hbm.at[ib], rb, sem).wait()
            pltpu.sync_copy(rb, o_hbm.at[pl.ds(base, _CHUNK)])

    return k(table, idxp)


def _sc_scatter_add(m, dstp, zeros_acc):
    epad, width = m.shape
    nacc = zeros_acc.shape[0]
    per_core = epad // _NC
    rows_per_tile = per_core // _NS
    nchunks = rows_per_tile // _CHUNK
    rpt_acc = nacc // _NS
    mesh = plsc.VectorSubcoreMesh(core_axis_name="c", subcore_axis_name="s")

    @functools.partial(
        pl.kernel,
        out_type=jax.ShapeDtypeStruct((2 * nacc, width), jnp.float32),
        mesh=mesh,
        scratch_types=[pltpu.VMEM((_CHUNK,), jnp.int32),
                       pltpu.VMEM((_CHUNK, width), jnp.float32),
                       pltpu.VMEM_SHARED((nacc, width), jnp.float32),
                       pltpu.SemaphoreType.DMA],
    )
    def k(m_hbm, dst_hbm, z_hbm, o_hbm, ib, mb, acc, sem):
        cid = lax.axis_index("c")
        sid = lax.axis_index("s")
        arow = sid * rpt_acc
        pltpu.sync_copy(z_hbm.at[pl.ds(arow, rpt_acc)],
                        acc.at[pl.ds(arow, rpt_acc)])
        plsc.subcore_barrier()
        base0 = cid * per_core + sid * rows_per_tile

        @pl.loop(0, nchunks)
        def _(i):
            base = base0 + i * _CHUNK
            pltpu.sync_copy(dst_hbm.at[pl.ds(base, _CHUNK)], ib)
            pltpu.sync_copy(m_hbm.at[pl.ds(base, _CHUNK)], mb)
            pltpu.sync_copy(mb, acc.at[ib], add=True)

        plsc.subcore_barrier()
        pltpu.sync_copy(acc.at[pl.ds(arow, rpt_acc)],
                        o_hbm.at[pl.ds(cid * nacc + arow, rpt_acc)])

    return k(m, dstp, zeros_acc)


def _sc_scatter4(ms, dstp, zeros_acc):
    epad, width = ms[0].shape
    nacc = zeros_acc.shape[0]
    rows_per_tile = epad // _NS
    nchunks = rows_per_tile // _CHUNK
    rpt_acc = nacc // _NS
    mesh = plsc.VectorSubcoreMesh(core_axis_name="c", subcore_axis_name="s")

    @functools.partial(
        pl.kernel,
        out_type=tuple(jax.ShapeDtypeStruct((nacc, width), jnp.float32)
                       for _ in range(_H)),
        mesh=mesh,
        scratch_types=[pltpu.VMEM((_CHUNK,), jnp.int32),
                       pltpu.VMEM((_CHUNK, width), jnp.float32),
                       pltpu.VMEM_SHARED((nacc, width), jnp.float32),
                       pltpu.SemaphoreType.DMA],
    )
    def k(m0, m1, m2, m3, dst_hbm, z_hbm, o0, o1, o2, o3, ib, mb, acc, sem):
        cid = lax.axis_index("c")
        sid = lax.axis_index("s")
        arow = sid * rpt_acc
        base0 = sid * rows_per_tile
        mrefs = (m0, m1, m2, m3)
        orefs = (o0, o1, o2, o3)

        for c in range(2):
            for hh in range(2):
                h = 2 * c + hh
                m_hbm = mrefs[h]
                o_hbm = orefs[h]

                @pl.when(cid == c)
                def _():
                    pltpu.sync_copy(z_hbm.at[pl.ds(arow, rpt_acc)],
                                    acc.at[pl.ds(arow, rpt_acc)])
                    plsc.subcore_barrier()

                    @pl.loop(0, nchunks)
                    def _(i):
                        base = base0 + i * _CHUNK
                        pltpu.sync_copy(dst_hbm.at[pl.ds(base, _CHUNK)], ib)
                        pltpu.sync_copy(m_hbm.at[pl.ds(base, _CHUNK)], mb)
                        pltpu.sync_copy(mb, acc.at[ib], add=True)

                    plsc.subcore_barrier()
                    pltpu.sync_copy(acc.at[pl.ds(arow, rpt_acc)],
                                    o_hbm.at[pl.ds(arow, rpt_acc)])

    return k(*ms, dstp, zeros_acc)


def _tc_scale4(xj, xi, u16p, cp, degcol):
    epad, width = xj.shape
    nhalf = width // _DX
    blk = 2048
    grid = epad // blk
    outs = tuple(jax.ShapeDtypeStruct((epad, _DX), jnp.float32)
                 for _ in range(_H * nhalf))

    def body(xj_ref, xi_ref, u_ref, c_ref, *o_refs):
        xjv = xj_ref[...]
        t16 = jnp.dot(xjv - xi_ref[...], u_ref[...],
                      preferred_element_type=jnp.float32)
        ts = [t16[:, h:h + 1] + c_ref[0:1, h:h + 1] for h in range(_H)]
        mx = jnp.maximum(jnp.maximum(ts[0], ts[1]), jnp.maximum(ts[2], ts[3]))
        es = [jnp.exp(t - mx) for t in ts]
        ssum = (es[0] + es[2]) + (es[1] + es[3])
        colid = lax.broadcasted_iota(jnp.int32, (blk, _DX), 1)
        for h in range(_H):
            q = es[h] / ssum
            for p in range(nhalf):
                m = q * xjv[:, p * _DX:(p + 1) * _DX]
                if h == 0 and degcol // _DX == p:
                    m = jnp.where(colid == degcol % _DX, 1.0, m)
                o_refs[h * nhalf + p][...] = m

    return pl.pallas_call(
        body,
        grid=(grid,),
        in_specs=[pl.BlockSpec((blk, width), lambda i: (i, 0)),
                  pl.BlockSpec((blk, width), lambda i: (i, 0)),
                  pl.BlockSpec((width, _DY), lambda i: (0, 0)),
                  pl.BlockSpec((1, _DY), lambda i: (0, 0))],
        out_specs=tuple(pl.BlockSpec((blk, _DX), lambda i: (i, 0))
                        for _ in range(_H * nhalf)),
        out_shape=outs,
    )(xj, xi, u16p, cp)


def _tc_res(x, pp):
    nacc, cxp = x.shape
    blk = 128

    def body(x_ref, p_ref, o_ref):
        o_ref[...] = jnp.dot(x_ref[...], p_ref[...],
                             preferred_element_type=jnp.float32)

    return pl.pallas_call(
        body,
        grid=(nacc // blk,),
        in_specs=[pl.BlockSpec((blk, cxp), lambda i: (i, 0)),
                  pl.BlockSpec((cxp, _DX), lambda i: (0, 0))],
        out_specs=pl.BlockSpec((blk, _DX), lambda i: (i, 0)),
        out_shape=jax.ShapeDtypeStruct((nacc, _DX), jnp.float32),
    )(x, pp)


def _tc_sumw(aggs, w4p, smallp, degcol):
    nacc, width = aggs[0].shape
    blk = 128

    def body(a0, a1, a2, a3, w_ref, sp_ref, o_ref):
        arefs = (a0, a1, a2, a3)
        s = jnp.zeros((blk, _DX), jnp.float32)
        for h in range(_H):
            s = s + jnp.dot(arefs[h][...],
                            w_ref[h * width:(h + 1) * width, :],
                            preferred_element_type=jnp.float32)
        deg = jnp.clip(a0[...][:, degcol:degcol + 1], 1.0, None)
        o_ref[...] = s / deg + sp_ref[0:1, :]

    return pl.pallas_call(
        body,
        grid=(nacc // blk,),
        in_specs=[pl.BlockSpec((blk, width), lambda i: (i, 0))] * 4
        + [pl.BlockSpec((_H * width, _DX), lambda i: (0, 0)),
           pl.BlockSpec((8, _DX), lambda i: (0, 0))],
        out_specs=pl.BlockSpec((blk, _DX), lambda i: (i, 0)),
        out_shape=jax.ShapeDtypeStruct((nacc, _DX), jnp.float32),
    )(*aggs, w4p, smallp)


def _tc_bnres(opre, res, smallp, n, batch_norm=True, relu=True,
              pool_rows=0, final=False):
    nacc = opre.shape[0]
    if final:
        outs = jax.ShapeDtypeStruct((nacc, 16), jnp.float32)
    elif pool_rows:
        outs = (jax.ShapeDtypeStruct((nacc, _DX), jnp.float32),
                jax.ShapeDtypeStruct((pool_rows, _DX), jnp.float32))
    else:
        outs = jax.ShapeDtypeStruct((nacc, _DX), jnp.float32)

    def body(p_ref, r_ref, sp_ref, o_ref, *maybe_pool):
        if final:
            o_ref[...] = p_ref[:, 0:16] + r_ref[0:nacc, 0:16]
            return
        o = p_ref[:, 0:_C]
        rowid = lax.broadcasted_iota(jnp.int32, (nacc, 1), 0)
        rmask = rowid < n
        if batch_norm:
            om = jnp.where(rmask, o, 0.0)
            mu = jnp.sum(om, axis=0, keepdims=True) / n
            d = o - mu
            dm = jnp.where(rmask, d, 0.0)
            var = jnp.sum(dm * dm, axis=0, keepdims=True) / n
            o = d / jnp.sqrt(var + 1e-5) * sp_ref[1:2, 0:_C] \
                + sp_ref[2:3, 0:_C]
        if relu:
            o = jnp.maximum(o, 0.0)
        o96 = jnp.concatenate([o, jnp.zeros((nacc, _DX - _C), jnp.float32)],
                              axis=1)
        xo = jnp.where(rmask, o96 + r_ref[...], 0.0)
        o_ref[...] = xo
        if pool_rows:
            colid = lax.broadcasted_iota(jnp.int32, (nacc, _DX), 1)
            xp = jnp.where(colid == 94, jnp.where(rmask, 1.0, 0.0), xo)
            maybe_pool[0][0:nacc, :] = xp
            maybe_pool[0][nacc:pool_rows, :] = jnp.zeros(
                (pool_rows - nacc, _DX), jnp.float32)

    return pl.pallas_call(body, out_shape=outs)(opre, res, smallp)


def _tc_pool_prep(parts, n):
    nacc2, width = parts.shape
    nacc = nacc2 // 2

    def body(p_ref, o_ref):
        agg = p_ref[0:nacc, :] + p_ref[nacc:2 * nacc, :]
        cnt = jnp.clip(agg[:, 94:95], 1.0, None)
        xm = agg[:, 0:_C] / cnt
        rowid = lax.broadcasted_iota(jnp.int32, (nacc, 1), 0)
        x96 = jnp.concatenate([xm, jnp.zeros((nacc, _DX - _C), jnp.float32)],
                              axis=1)
        o_ref[...] = jnp.where(rowid < n, x96, 0.0)

    return pl.pallas_call(
        body,
        out_shape=jax.ShapeDtypeStruct((nacc, _DX), jnp.float32),
    )(parts)


def _tc_concat(xup, copy, n):
    nacc = copy.shape[0]

    def body(xu_ref, cp_ref, o_ref):
        xu = xu_ref[...]
        cp = cp_ref[...]
        rowid = lax.broadcasted_iota(jnp.int32, (nacc, 1), 0)
        cat = jnp.concatenate(
            [xu[:, 0:_C], cp[:, 0:_C],
             jnp.zeros((nacc, 256 - 2 * _C), jnp.float32)], axis=1)
        o_ref[...] = jnp.where(rowid < n, cat, 0.0)

    return pl.pallas_call(
        body,
        out_shape=jax.ShapeDtypeStruct((nacc, 256), jnp.float32),
    )(xup[:nacc], copy)


def _pack_params(p, in_c, out_c, in_pad):
    w = p['W']
    u = p['u']
    w4 = jnp.zeros((_H * in_pad, _DX), jnp.float32)
    for h in range(_H):
        w4 = w4.at[h * in_pad:h * in_pad + in_c, :out_c].set(w[h])
    u16 = jnp.zeros((in_pad, _DY), jnp.float32)
    u16 = u16.at[:in_c, 0:_H].set(u)
    pp = jnp.zeros((in_pad, _DX), jnp.float32)
    pp = pp.at[:in_c, :out_c].set(p['P'])
    sp = jnp.zeros((8, _DX), jnp.float32)
    sp = sp.at[0, :out_c].set(p['b'])
    sp = sp.at[1, :out_c].set(p['gamma'])
    sp = sp.at[2, :out_c].set(p['beta'])
    cp = jnp.zeros((1, _DY), jnp.float32)
    cp = cp.at[0, 0:_H].set(p['c'])
    return w4, u16, pp, sp, cp


def _pad_idx(idx, total, dummy):
    idx = idx.astype(jnp.int32)
    return jnp.concatenate(
        [idx, jnp.full((total - idx.shape[0],), dummy, jnp.int32)])


def _pad_rows(x, rows, width):
    out = jnp.zeros((rows, width), jnp.float32)
    return out.at[:x.shape[0], :x.shape[1]].set(x)


def kernel(norm, geo, edge_index0, cluster1, edge_index1, cluster2,
           edge_index2, params):
    e0p = _ceil_to(_E0, _EALIGN)
    e1p = _ceil_to(_E1, _EALIGN)
    e2p = _ceil_to(_E2, _EALIGN)
    p0p = _ceil_to(_N0, _EALIGN)
    p1p = _ceil_to(_N1, _EALIGN)

    def _sorted_pair(ei, epad, n):
        perm = jnp.argsort(ei[1], stable=True)
        return (_pad_idx(ei[0][perm], epad, n),
                _pad_idx(ei[1][perm], epad, n))

    src0, dst0 = _sorted_pair(edge_index0, e0p, _N0)
    src1, dst1 = _sorted_pair(edge_index1, e1p, _N1)
    src2, dst2 = _sorted_pair(edge_index2, e2p, _N2)
    cl1 = _pad_idx(cluster1, p0p, _N1)
    cl2 = _pad_idx(cluster2, p1p, _N2)

    z0 = jnp.zeros((_N0A, _DX), jnp.float32)
    z1 = jnp.zeros((_N1A, _DX), jnp.float32)
    z2 = jnp.zeros((_N2A, _DX), jnp.float32)

    specs = {'conv01': 4, 'conv02': _C, 'conv11': _C, 'conv12': _C,
             'conv21': _C, 'conv22': _C, 'conv13': 2 * _C, 'conv14': _C,
             'conv15': _C, 'conv16': _C, 'conv03': 2 * _C, 'conv04': _C,
             'conv05': _C, 'conv06': _C}
    inpad = {4: _DX, _C: _DX, 2 * _C: 256}
    pk = {}
    for name, ic in specs.items():
        oc = 3 if name == 'conv06' else _C
        pk[name] = _pack_params(params[name], ic, oc, inpad[ic])

    def conv(name, x_in, srcp, dstp, zacc, n, batch_norm=True, relu=True,
             pool_rows=0, final=False):
        w4, u16, pp, sp, cp = pk[name]
        width = x_in.shape[1]
        nhalf = width // _DX
        degcol = 94 if nhalf == 1 else 192
        xj, xi = _sc_gather_two(x_in, srcp, dstp)
        ms = _tc_scale4(xj, xi, u16, cp, degcol)
        if nhalf == 1:
            aggs = _sc_scatter4(ms, dstp, zacc)
        else:
            pa = _sc_scatter4(ms[0::2], dstp, zacc)
            pb = _sc_scatter4(ms[1::2], dstp, zacc)
            aggs = tuple(jnp.concatenate([pa[h], pb[h]], axis=1)
                         for h in range(_H))
        opre = _tc_sumw(aggs, w4, sp, degcol)
        res = _tc_res(x_in, pp)
        return _tc_bnres(opre, res, sp, n, batch_norm, relu, pool_rows, final)

    x0 = _pad_rows(jnp.concatenate([norm, geo[:, None]], axis=1), _N0A, _DX)
    x = conv('conv01', x0, src0, dst0, z0, _N0)
    x, xpool = conv('conv02', x, src0, dst0, z0, _N0, pool_rows=p0p)
    copy0 = x
    parts1 = _sc_scatter_add(xpool, cl1, z1)
    x1 = _tc_pool_prep(parts1, _N1)
    x1 = conv('conv11', x1, src1, dst1, z1, _N1)
    x1, xpool1 = conv('conv12', x1, src1, dst1, z1, _N1, pool_rows=p1p)
    copy1 = x1
    parts2 = _sc_scatter_add(xpool1, cl2, z2)
    x2 = _tc_pool_prep(parts2, _N2)
    x2 = conv('conv21', x2, src2, dst2, z2, _N2)
    x2 = conv('conv22', x2, src2, dst2, z2, _N2)
    x1up = _sc_gather_one(x2, cl2)
    x1c = _tc_concat(x1up, copy1, _N1)
    x1 = conv('conv13', x1c, src1, dst1, z1, _N1)
    x1 = conv('conv14', x1, src1, dst1, z1, _N1)
    x1 = conv('conv15', x1, src1, dst1, z1, _N1)
    x1 = conv('conv16', x1, src1, dst1, z1, _N1)
    x0up = _sc_gather_one(x1, cl1)
    x0c = _tc_concat(x0up, copy0, _N0)
    x = conv('conv03', x0c, src0, dst0, z0, _N0)
    x = conv('conv04', x, src0, dst0, z0, _N0)
    x = conv('conv05', x, src0, dst0, z0, _N0)
    out = conv('conv06', x, src0, dst0, z0, _N0, final=True)
    return out[:_N0, :3]
